# baseline (speedup 1.0000x reference)
import functools

import jax
import jax.numpy as jnp
from jax import lax
from jax.experimental import pallas as pl
from jax.experimental.pallas import tpu as pltpu
from jax.experimental.pallas import tpu_sc as plsc

NUM_FIELDS = 26
VOCAB = 100000
VOCAB_PAD = 100096
NTILE = VOCAB_PAD // 128
EMBED_DIM = 16
BATCH = 16384

NUM_WORKERS = 32
UPF = 196
UNITS = NUM_FIELDS * UPF
UNITS_PW = 160

BPW = BATCH // NUM_WORKERS
ROWS_PW = BPW * NUM_FIELDS
IDX_ROWS = ROWS_PW // 128
CHUNK = 64
CROWS = CHUNK * NUM_FIELDS
CSUB = CROWS // 128
NCHUNK = BPW // CHUNK
GROUPS = CHUNK // 16

L_ROWS = NUM_FIELDS * VOCAB_PAD


def _make_transpose_kernel():
    mesh = plsc.VectorSubcoreMesh(core_axis_name="c", subcore_axis_name="s")

    @functools.partial(
        pl.kernel,
        mesh=mesh,
        compiler_params=pltpu.CompilerParams(
            needs_layout_passes=False, use_tc_tiling_on_sc=True),
        out_type=jax.ShapeDtypeStruct((L_ROWS * EMBED_DIM,), jnp.float32),
        scratch_types=(
            [pltpu.VMEM((16, 512), jnp.float32)] * 2
            + [pltpu.VMEM((8192,), jnp.float32)] * 2
            + [pltpu.SemaphoreType.DMA, pltpu.SemaphoreType.DMA]
        ),
    )
    def detile(a_hbm, l_hbm, ba, bb, oa, ob, sem_in, sem_out):
        wid = lax.axis_index("s") * 2 + lax.axis_index("c")
        scat = lax.iota(jnp.int32, 16) * 16
        bufs = ((ba, oa), (bb, ob))

        def fcol(k):
            u = jnp.minimum(k * NUM_WORKERS + wid, UNITS - 1)
            f = u // UPF
            col0 = jnp.minimum(512 * (u % UPF), VOCAB_PAD - 512)
            return f, col0

        def issue_in(k, par):
            f, col0 = fcol(k)
            pltpu.async_copy(
                a_hbm.at[pl.ds(16 * f, 16), pl.ds(col0, 512)],
                bufs[par][0], sem_in)

        def wait_in(par):
            pltpu.make_async_copy(
                a_hbm.at[pl.ds(0, 16), pl.ds(0, 512)], bufs[par][0],
                sem_in).wait()

        def wait_out(par):
            pltpu.make_async_copy(
                l_hbm.at[pl.ds(0, 8192)], bufs[par][1], sem_out).wait()

        issue_in(0, 0)
        issue_in(1, 1)

        def pair_body(kk, _):
            for par in range(2):
                k = kk * 2 + par
                b, obuf = bufs[par]
                wait_in(par)

                @pl.when(kk > 0)
                def _():
                    wait_out(par)

                for d in range(16):
                    idx_d = scat + d
                    vs = [b[d, pl.ds(cw * 16, 16)] for cw in range(32)]
                    for cw in range(32):
                        tgt = obuf.at[pl.ds(cw * 256, 256)]
                        plsc.store_scatter(tgt, [idx_d], vs[cw])
                f, col0 = fcol(k)
                base = (f * VOCAB_PAD + col0) * EMBED_DIM
                pltpu.async_copy(
                    obuf, l_hbm.at[pl.ds(base, 8192)], sem_out)

                @pl.when(k + 2 < UNITS_PW)
                def _():
                    issue_in(k + 2, par)
            return _

        lax.fori_loop(0, UNITS_PW // 2, pair_body, None)
        wait_out(0)
        wait_out(1)

    return detile


def _make_fm_kernel():
    mesh = plsc.VectorSubcoreMesh(core_axis_name="c", subcore_axis_name="s")

    @functools.partial(
        pl.kernel,
        mesh=mesh,
        compiler_params=pltpu.CompilerParams(
            needs_layout_passes=False, use_tc_tiling_on_sc=False),
        out_type=jax.ShapeDtypeStruct((BATCH,), jnp.float32),
        scratch_types=[
            pltpu.VMEM((IDX_ROWS, 128), jnp.int32),
            pltpu.VMEM((IDX_ROWS, 128), jnp.int32),
            pltpu.VMEM((CROWS, EMBED_DIM), jnp.float32),
            pltpu.VMEM((CROWS + 32,), jnp.float32),
            pltpu.VMEM((BPW,), jnp.float32),
            pltpu.VMEM((16,), jnp.float32),
            pltpu.SemaphoreType.DMA,
            pltpu.SemaphoreType.DMA,
        ],
    )
    def fm_sc(xg_hbm, xl_hbm, intf_hbm, linf_hbm, bias_hbm, out_hbm,
              x_v, xl_v, rows_v, lin_v, out_v, bias_v, sem_i, sem_l):
        wid = lax.axis_index("s") * 2 + lax.axis_index("c")
        pltpu.sync_copy(xg_hbm.at[pl.ds(wid * IDX_ROWS, IDX_ROWS)], x_v)
        pltpu.sync_copy(xl_hbm.at[pl.ds(wid * IDX_ROWS, IDX_ROWS)], xl_v)
        pltpu.sync_copy(bias_hbm, bias_v)

        lanes = lax.iota(jnp.int32, 16)
        lin_mask = jnp.where(lanes < (NUM_FIELDS - 16), 1.0, 0.0)

        def chunk_body(c, _):
            copies = []
            for j in range(CSUB):
                copies.append(pltpu.async_copy(
                    intf_hbm.at[x_v.at[c * CSUB + j]],
                    rows_v.at[pl.ds(j * 128, 128)], sem_i))
                copies.append(pltpu.async_copy(
                    linf_hbm.at[xl_v.at[c * CSUB + j]],
                    lin_v.at[pl.ds(j * 128, 128)], sem_l))
            for cp in copies:
                cp.wait()

            def group_body(g, _):
                res = jnp.zeros((16,), jnp.float32)
                for i in range(16):
                    item = g * 16 + i
                    base = item * NUM_FIELDS
                    v = rows_v[base]
                    s = v
                    q = v * v
                    for f in range(1, NUM_FIELDS):
                        v = rows_v[base + f]
                        s = s + v
                        q = q + v * v
                    l1 = lin_v[pl.ds(base, 16)]
                    l2 = lin_v[pl.ds(base + 16, 16)]
                    p = 0.5 * (s * s - q) + l1 + l2 * lin_mask
                    res = jnp.where(lanes == i, jnp.sum(p), res)
                out_v[pl.ds(c * CHUNK + g * 16, 16)] = res + bias_v[...]
                return _

            lax.fori_loop(0, GROUPS, group_body, None)
            return _

        lax.fori_loop(0, NCHUNK, chunk_body, None)
        pltpu.sync_copy(out_v, out_hbm.at[pl.ds(wid * BPW, BPW)])

    return fm_sc


_DETILE = _make_transpose_kernel()
_FM_SC = _make_fm_kernel()


@jax.jit
def kernel(X, int_tables, lin_tables, bias):
    a_view = jnp.transpose(int_tables, (0, 2, 1)).reshape(
        NUM_FIELDS * EMBED_DIM, VOCAB)
    l_flat = _DETILE(a_view)
    intf = l_flat.reshape(L_ROWS, EMBED_DIM)

    offs = jnp.arange(NUM_FIELDS, dtype=jnp.int32) * VOCAB_PAD
    xg = (X.astype(jnp.int32) + offs[None, :]).reshape(
        NUM_WORKERS * IDX_ROWS, 128)
    loffs = jnp.arange(NUM_FIELDS, dtype=jnp.int32) * VOCAB
    xl = (X.astype(jnp.int32) + loffs[None, :]).reshape(
        NUM_WORKERS * IDX_ROWS, 128)
    linf = lin_tables.reshape(NUM_FIELDS * VOCAB)
    bias16 = jnp.broadcast_to(bias.reshape(()), (16,)).astype(jnp.float32)
    out = _FM_SC(xg, xl, intf, linf, bias16)
    return out.reshape(BATCH, 1)


# speedup vs baseline: 1.4417x; 1.4417x over previous
import functools

import jax
import jax.numpy as jnp
from jax import lax
from jax.experimental import pallas as pl
from jax.experimental.pallas import tpu as pltpu
from jax.experimental.pallas import tpu_sc as plsc

NUM_FIELDS = 26
VOCAB = 100000
VOCAB_PAD = 100096
NTILE = VOCAB_PAD // 128
EMBED_DIM = 16
BATCH = 16384

NUM_WORKERS = 32
NTILE2 = NTILE // 2
UNITS = NUM_FIELDS * NTILE2
UNITS_PW = 318

BPW = BATCH // NUM_WORKERS
ROWS_PW = BPW * NUM_FIELDS
IDX_ROWS = ROWS_PW // 128
CHUNK = 64
CROWS = CHUNK * NUM_FIELDS
CSUB = CROWS // 128
NCHUNK = BPW // CHUNK
GROUPS = CHUNK // 16

L_ROWS = NUM_FIELDS * VOCAB_PAD


def _make_transpose_kernel():
    mesh = plsc.VectorSubcoreMesh(core_axis_name="c", subcore_axis_name="s")

    @functools.partial(
        pl.kernel,
        mesh=mesh,
        compiler_params=pltpu.CompilerParams(
            needs_layout_passes=False, use_tc_tiling_on_sc=True),
        out_type=jax.ShapeDtypeStruct((L_ROWS * EMBED_DIM,), jnp.float32),
        scratch_types=(
            [pltpu.VMEM((8, 256), jnp.float32)] * 4
            + [pltpu.VMEM((4096,), jnp.float32)] * 2
            + [pltpu.SemaphoreType.DMA, pltpu.SemaphoreType.DMA]
        ),
    )
    def detile(a_hbm, l_hbm, b0a, b0b, b1a, b1b, oa, ob, sem_in, sem_out):
        wid = lax.axis_index("s") * 2 + lax.axis_index("c")
        scat = lax.iota(jnp.int32, 16) * 16
        bufs = ((b0a, b1a, oa), (b0b, b1b, ob))

        def fj(k):
            u = jnp.minimum(k * NUM_WORKERS + wid, UNITS - 1)
            return u // NTILE2, u % NTILE2

        def issue_in(k, par):
            f, j = fj(k)
            b0, b1, _ = bufs[par]
            pltpu.async_copy(
                a_hbm.at[pl.ds(16 * f, 8), pl.ds(256 * j, 256)], b0, sem_in)
            pltpu.async_copy(
                a_hbm.at[pl.ds(16 * f + 8, 8), pl.ds(256 * j, 256)],
                b1, sem_in)

        def wait_in(par):
            b0, b1, _ = bufs[par]
            pltpu.make_async_copy(
                a_hbm.at[pl.ds(0, 8), pl.ds(0, 256)], b0, sem_in).wait()
            pltpu.make_async_copy(
                a_hbm.at[pl.ds(0, 8), pl.ds(0, 256)], b1, sem_in).wait()

        def wait_out(par):
            pltpu.make_async_copy(
                l_hbm.at[pl.ds(0, 4096)], bufs[par][2], sem_out).wait()

        issue_in(0, 0)
        issue_in(1, 1)

        def pair_body(kk, _):
            for par in range(2):
                k = kk * 2 + par
                b0, b1, obuf = bufs[par]
                wait_in(par)

                @pl.when(kk > 0)
                def _():
                    wait_out(par)

                for d in range(8):
                    idx_lo = scat + d
                    idx_hi = scat + (d + 8)
                    v0s = [b0[d, pl.ds(cw * 16, 16)] for cw in range(16)]
                    v1s = [b1[d, pl.ds(cw * 16, 16)] for cw in range(16)]
                    for cw in range(16):
                        tgt = obuf.at[pl.ds(cw * 256, 256)]
                        plsc.store_scatter(tgt, [idx_lo], v0s[cw])
                        plsc.store_scatter(tgt, [idx_hi], v1s[cw])
                f, j = fj(k)
                base = (f * VOCAB_PAD + 256 * j) * EMBED_DIM
                pltpu.async_copy(
                    obuf, l_hbm.at[pl.ds(base, 4096)], sem_out)

                @pl.when(k + 2 < UNITS_PW)
                def _():
                    issue_in(k + 2, par)
            return _

        lax.fori_loop(0, UNITS_PW // 2, pair_body, None)
        wait_out(0)
        wait_out(1)

    return detile


def _make_fm_kernel():
    mesh = plsc.VectorSubcoreMesh(core_axis_name="c", subcore_axis_name="s")

    @functools.partial(
        pl.kernel,
        mesh=mesh,
        compiler_params=pltpu.CompilerParams(
            needs_layout_passes=False, use_tc_tiling_on_sc=False),
        out_type=jax.ShapeDtypeStruct((BATCH,), jnp.float32),
        scratch_types=(
            [pltpu.VMEM((IDX_ROWS, 128), jnp.int32)] * 2
            + [pltpu.VMEM((CROWS, EMBED_DIM), jnp.float32)] * 2
            + [pltpu.VMEM((CROWS + 32,), jnp.float32)] * 2
            + [pltpu.VMEM((BPW,), jnp.float32),
               pltpu.VMEM((16,), jnp.float32),
               pltpu.SemaphoreType.DMA,
               pltpu.SemaphoreType.DMA]
        ),
    )
    def fm_sc(xg_hbm, xl_hbm, intf_hbm, linf_hbm, bias_hbm, out_hbm,
              x_v, xl_v, rows_a, rows_b, lin_a, lin_b,
              out_v, bias_v, sem_i, sem_l):
        wid = lax.axis_index("s") * 2 + lax.axis_index("c")
        pltpu.sync_copy(xg_hbm.at[pl.ds(wid * IDX_ROWS, IDX_ROWS)], x_v)
        pltpu.sync_copy(xl_hbm.at[pl.ds(wid * IDX_ROWS, IDX_ROWS)], xl_v)
        pltpu.sync_copy(bias_hbm, bias_v)

        lanes = lax.iota(jnp.int32, 16)
        lin_mask = jnp.where(lanes < (NUM_FIELDS - 16), 1.0, 0.0)
        bufs = ((rows_a, lin_a), (rows_b, lin_b))

        def issue(c, par):
            rows_v, lin_v = bufs[par]
            for j in range(CSUB):
                pltpu.async_copy(
                    intf_hbm.at[x_v.at[c * CSUB + j]],
                    rows_v.at[pl.ds(j * 128, 128)], sem_i)
                pltpu.async_copy(
                    linf_hbm.at[xl_v.at[c * CSUB + j]],
                    lin_v.at[pl.ds(j * 128, 128)], sem_l)

        def drain(par):
            rows_v, lin_v = bufs[par]
            for j in range(CSUB):
                pltpu.make_async_copy(
                    intf_hbm.at[pl.ds(0, 128)],
                    rows_v.at[pl.ds(j * 128, 128)], sem_i).wait()
                pltpu.make_async_copy(
                    linf_hbm.at[pl.ds(0, 128)],
                    lin_v.at[pl.ds(j * 128, 128)], sem_l).wait()

        issue(0, 0)
        issue(1, 1)

        def chunk_pair(cc, _):
            for par in range(2):
                c = cc * 2 + par
                rows_v, lin_v = bufs[par]
                drain(par)

                def group_body(g, _):
                    res = jnp.zeros((16,), jnp.float32)
                    for i in range(16):
                        item = g * 16 + i
                        base = item * NUM_FIELDS
                        v = rows_v[base]
                        s = v
                        q = v * v
                        for f in range(1, NUM_FIELDS):
                            v = rows_v[base + f]
                            s = s + v
                            q = q + v * v
                        l1 = lin_v[pl.ds(base, 16)]
                        l2 = lin_v[pl.ds(base + 16, 16)]
                        p = 0.5 * (s * s - q) + l1 + l2 * lin_mask
                        res = jnp.where(lanes == i, jnp.sum(p), res)
                    out_v[pl.ds(c * CHUNK + g * 16, 16)] = res + bias_v[...]
                    return _

                lax.fori_loop(0, GROUPS, group_body, None)

                @pl.when(c + 2 < NCHUNK)
                def _():
                    issue(c + 2, par)
            return _

        lax.fori_loop(0, NCHUNK // 2, chunk_pair, None)
        pltpu.sync_copy(out_v, out_hbm.at[pl.ds(wid * BPW, BPW)])

    return fm_sc


_DETILE = _make_transpose_kernel()
_FM_SC = _make_fm_kernel()


@jax.jit
def kernel(X, int_tables, lin_tables, bias):
    a_view = jnp.transpose(int_tables, (0, 2, 1)).reshape(
        NUM_FIELDS * EMBED_DIM, VOCAB)
    l_flat = _DETILE(a_view)
    intf = l_flat.reshape(L_ROWS, EMBED_DIM)

    offs = jnp.arange(NUM_FIELDS, dtype=jnp.int32) * VOCAB_PAD
    xg = (X.astype(jnp.int32) + offs[None, :]).reshape(
        NUM_WORKERS * IDX_ROWS, 128)
    loffs = jnp.arange(NUM_FIELDS, dtype=jnp.int32) * VOCAB
    xl = (X.astype(jnp.int32) + loffs[None, :]).reshape(
        NUM_WORKERS * IDX_ROWS, 128)
    linf = lin_tables.reshape(NUM_FIELDS * VOCAB)
    bias16 = jnp.broadcast_to(bias.reshape(()), (16,)).astype(jnp.float32)
    out = _FM_SC(xg, xl, intf, linf, bias16)
    return out.reshape(BATCH, 1)
